# baseline (speedup 1.0000x reference)
import functools

import jax
import jax.numpy as jnp
from jax import lax
from jax.experimental import pallas as pl
from jax.experimental.pallas import tpu as pltpu
from jax.experimental.pallas import tpu_sc as plsc

N = 10000
E = 320000
H = 128
DT = 0.03

C = 4
EC = E // C
GW = 128
BE = 640
PAD = 1920
RC = 400
NCH = N // RC

_MESH = dict(core_axis_name="core", subcore_axis_name="subcore")


GR = 4
GL = 2
WPT = 40
MP = 32 * WPT * GW


def _sc_gather(h, idx_flat):
    mesh = plsc.VectorSubcoreMesh(**_MESH)

    @functools.partial(
        pl.kernel,
        out_type=jax.ShapeDtypeStruct((MP, H), jnp.float32),
        mesh=mesh,
        scratch_types=[
            pltpu.VMEM((WPT * GW,), jnp.int32),
            pltpu.VMEM((GR, GW, H), jnp.float32),
            pltpu.SemaphoreType.DMA((GR,)),
            pltpu.SemaphoreType.DMA((GR,)),
        ],
    )
    def k(h_hbm, i_hbm, o_hbm, idx_all, rows, gsem, wsem):
        wid = lax.axis_index("core") * 16 + lax.axis_index("subcore")
        base_e = wid * (WPT * GW)

        pltpu.sync_copy(i_hbm.at[0, pl.ds(base_e, WPT * GW)], idx_all)

        def gth(w, d):
            return pltpu.make_async_copy(
                h_hbm.at[idx_all.at[pl.ds(w * GW, GW)]], rows.at[d],
                gsem.at[d])

        def wb(w, d):
            return pltpu.make_async_copy(
                rows.at[d], o_hbm.at[pl.ds((base_e + w * GW), GW), :],
                wsem.at[d])

        for d in range(GL):
            gth(d, d).start()

        @pl.loop(0, WPT, step=GR)
        def _(w0):
            for d in range(GR):
                w = w0 + d
                gth(w, d).wait()
                wb(w, d).start()
                wn = w + GL
                s = (d + GL) % GR

                @pl.when(wn < WPT)
                def _():
                    @pl.when(wn >= GR)
                    def _():
                        wb(wn - GR, s).wait()

                    gth(wn, s).start()

        for d in range(GR):
            wb(WPT - GR + d, d).wait()

    return k(h, idx_flat)


def _tc_mlp(gath, W1, b1, w2v, b2v, W2bd, b2pf):
    nblk = EC // BE
    coff = (MP // 2) // BE

    def body(xr_ref, xc_ref, W1_ref, b1_ref, w2v_ref, b2v_ref, W2_ref,
             b2pf_ref, o_ref):
        xr = xr_ref[...]
        xc = xc_ref[...]
        x = jnp.concatenate([xr, xc], axis=1).astype(jnp.bfloat16)
        z = jnp.dot(x, W1_ref[...],
                    preferred_element_type=jnp.float32) + b1_ref[...]
        t_vp = jnp.tanh(z[:, :256])
        t_f = jnp.maximum(z[:, 256:], 0.0)
        nu = (jnp.sum(t_vp[:, :128] * w2v_ref[...], axis=1, keepdims=True)
              + b2v_ref[0, 0])
        tpf = jnp.concatenate([t_vp[:, 128:], t_f],
                              axis=1).astype(jnp.bfloat16)
        pf = jnp.dot(tpf, W2_ref[...],
                     preferred_element_type=jnp.float32) + b2pf_ref[...]
        o_ref[...] = nu * (xc - xr) + pf[:, 128:] - pf[:, :128]

    full = lambda shape: pl.BlockSpec(shape, lambda i: (0, 0))
    return pl.pallas_call(
        body,
        grid=(nblk,),
        in_specs=[
            pl.BlockSpec((BE, H), lambda i: (i, 0)),
            pl.BlockSpec((BE, H), lambda i: (i + coff, 0)),
            full((256, 384)),
            full((1, 384)),
            full((1, 128)),
            full((1, 1)),
            full((256, 256)),
            full((1, 256)),
        ],
        out_specs=pl.BlockSpec((BE, H), lambda i: (i, 0)),
        out_shape=jax.ShapeDtypeStruct((EC, H), jnp.float32),
    )(gath, gath, W1, b1, w2v, b2v, W2bd, b2pf)


def _sc_scatter(msg_chunks, idx_chunks, zrows):
    mesh = plsc.VectorSubcoreMesh(**_MESH)

    @functools.partial(
        pl.kernel,
        out_type=jax.ShapeDtypeStruct((2, N, H), jnp.float32),
        mesh=mesh,
        scratch_types=[pltpu.VMEM_SHARED((N, H), jnp.float32)],
    )
    def k(*refs):
        m_refs = refs[:C]
        i_refs = refs[C:2 * C]
        z_hbm = refs[2 * C]
        o_hbm = refs[2 * C + 1]
        acc = refs[2 * C + 2]
        sid = lax.axis_index("subcore")
        cid = lax.axis_index("core")

        @pl.loop(0, 2)
        def _(j):
            c = j * 16 + sid

            @pl.when(c < NCH)
            def _():
                pltpu.sync_copy(z_hbm, acc.at[pl.ds(c * RC, RC), :])

        plsc.subcore_barrier()

        def body(m_vmem, i_vmem):
            pltpu.sync_copy(m_vmem, acc.at[i_vmem.at[0]], add=True)

        for c in range(C):
            pltpu.emit_pipeline(
                body,
                grid=(EC // GW,),
                in_specs=[
                    pl.BlockSpec((GW, H), lambda i: (i, 0)),
                    pl.BlockSpec((1, GW), lambda i: (0, i)),
                ],
                out_specs=[],
                core_axis_name=("core", "subcore"),
                dimension_semantics=(pltpu.PARALLEL,),
            )(m_refs[c], i_refs[c])
        plsc.subcore_barrier()

        @pl.loop(0, 2)
        def _(j):
            c = j * 16 + sid

            @pl.when(c < NCH)
            def _():
                pltpu.sync_copy(acc.at[pl.ds(c * RC, RC), :],
                                o_hbm.at[cid, pl.ds(c * RC, RC), :])

    return k(*msg_chunks, *idx_chunks, zrows)


def _tc_combine(h, parts):
    def body(h_ref, p_ref, o_ref):
        o_ref[...] = h_ref[...] + DT * (p_ref[0] + p_ref[1])

    return pl.pallas_call(
        body,
        grid=(10,),
        in_specs=[
            pl.BlockSpec((N // 10, H), lambda i: (i, 0)),
            pl.BlockSpec((2, N // 10, H), lambda i: (0, i, 0)),
        ],
        out_specs=pl.BlockSpec((N // 10, H), lambda i: (i, 0)),
        out_shape=jax.ShapeDtypeStruct((N, H), jnp.float32),
    )(h, parts)


def kernel(h, edge_index, v_w1, v_b1, v_w2, v_b2, p_w1, p_b1, p_w2, p_b2,
           f_w1, f_b1, f_w2, f_b2):
    rowc = edge_index[0].reshape(C, EC)
    colc = edge_index[1].reshape(C, EC)
    rowp = jnp.pad(rowc, ((0, 0), (0, PAD)))
    colp = jnp.pad(colc, ((0, 0), (0, PAD)))
    idx_pair = jnp.stack([rowp, colp], axis=1)

    W1 = jnp.concatenate([v_w1.T, p_w1.T, f_w1.T],
                         axis=1).astype(jnp.bfloat16)
    b1 = jnp.concatenate([v_b1, p_b1, f_b1]).reshape(1, 384)
    W2bd = jnp.zeros((256, 256), jnp.float32)
    W2bd = W2bd.at[:128, :128].set(p_w2.T).at[128:, 128:].set(f_w2.T)
    W2bd = W2bd.astype(jnp.bfloat16)
    b2pf = jnp.concatenate([p_b2, f_b2]).reshape(1, 256)
    w2v = v_w2.reshape(1, 128)
    b2v = v_b2.reshape(1, 1)
    zrows = jnp.zeros((RC, H), jnp.float32)

    msg_chunks = []
    idx_chunks = []
    for c in range(C):
        gath = _sc_gather(h, idx_pair[c].reshape(1, MP))
        msg_chunks.append(_tc_mlp(gath, W1, b1, w2v, b2v, W2bd, b2pf))
        idx_chunks.append(rowc[c:c + 1, :])

    parts = _sc_scatter(msg_chunks, idx_chunks, zrows)
    return _tc_combine(h, parts)


# speedup vs baseline: 2.3609x; 2.3609x over previous
import functools

import jax
import jax.numpy as jnp
from jax import lax
from jax.experimental import pallas as pl
from jax.experimental.pallas import tpu as pltpu
from jax.experimental.pallas import tpu_sc as plsc

N = 10000
E = 320000
H = 128
DT = 0.03

C = 4
EC = E // C
GW = 128
BE = 1600
RC = 400
NCH = N // RC

_MESH = dict(core_axis_name="core", subcore_axis_name="subcore")


def _sc_gather(h, idx_flat):
    M = idx_flat.shape[1]
    mesh = plsc.VectorSubcoreMesh(**_MESH)

    @functools.partial(
        pl.kernel,
        out_type=jax.ShapeDtypeStruct((M, H), jnp.float32),
        mesh=mesh,
        scratch_types=[pltpu.VMEM_SHARED((N, H), jnp.float32)],
    )
    def k(h_hbm, i_hbm, o_hbm, hs):
        sid = lax.axis_index("subcore")

        @pl.loop(0, 2)
        def _(j):
            c = j * 16 + sid

            @pl.when(c < NCH)
            def _():
                pltpu.sync_copy(h_hbm.at[pl.ds(c * RC, RC), :],
                                hs.at[pl.ds(c * RC, RC), :])

        plsc.subcore_barrier()

        def body(i_vmem, o_vmem):
            pltpu.sync_copy(hs.at[i_vmem.at[0]], o_vmem)

        pltpu.emit_pipeline(
            body,
            grid=(M // GW,),
            in_specs=[pl.BlockSpec((1, GW), lambda i: (0, i))],
            out_specs=[pl.BlockSpec((GW, H), lambda i: (i, 0))],
            core_axis_name=("core", "subcore"),
            dimension_semantics=(pltpu.PARALLEL,),
        )(i_hbm, o_hbm)

    return k(h, idx_flat)


def _tc_mlp(gath, W1, b1, w2v, b2v, W2bd, b2pf):
    nblk = EC // BE
    coff = nblk

    def body(xr_ref, xc_ref, W1_ref, b1_ref, w2v_ref, b2v_ref, W2_ref,
             b2pf_ref, o_ref):
        xr = xr_ref[...]
        xc = xc_ref[...]
        x = jnp.concatenate([xr, xc], axis=1).astype(jnp.bfloat16)
        z = jnp.dot(x, W1_ref[...],
                    preferred_element_type=jnp.float32) + b1_ref[...]
        t_vp = jnp.tanh(z[:, :256])
        t_f = jnp.maximum(z[:, 256:], 0.0)
        nu = (jnp.sum(t_vp[:, :128] * w2v_ref[...], axis=1, keepdims=True)
              + b2v_ref[0, 0])
        tpf = jnp.concatenate([t_vp[:, 128:], t_f],
                              axis=1).astype(jnp.bfloat16)
        pf = jnp.dot(tpf, W2_ref[...],
                     preferred_element_type=jnp.float32) + b2pf_ref[...]
        o_ref[...] = nu * (xc - xr) + pf[:, 128:] - pf[:, :128]

    full = lambda shape: pl.BlockSpec(shape, lambda i: (0, 0))
    return pl.pallas_call(
        body,
        grid=(nblk,),
        in_specs=[
            pl.BlockSpec((BE, H), lambda i: (i, 0)),
            pl.BlockSpec((BE, H), lambda i: (i + coff, 0)),
            full((256, 384)),
            full((1, 384)),
            full((1, 128)),
            full((1, 1)),
            full((256, 256)),
            full((1, 256)),
        ],
        out_specs=pl.BlockSpec((BE, H), lambda i: (i, 0)),
        out_shape=jax.ShapeDtypeStruct((EC, H), jnp.float32),
    )(gath, gath, W1, b1, w2v, b2v, W2bd, b2pf)


def _sc_scatter(msg_chunks, idx_chunks, zrows):
    mesh = plsc.VectorSubcoreMesh(**_MESH)

    @functools.partial(
        pl.kernel,
        out_type=jax.ShapeDtypeStruct((2, N, H), jnp.float32),
        mesh=mesh,
        scratch_types=[pltpu.VMEM_SHARED((N, H), jnp.float32)],
    )
    def k(*refs):
        m_refs = refs[:C]
        i_refs = refs[C:2 * C]
        z_hbm = refs[2 * C]
        o_hbm = refs[2 * C + 1]
        acc = refs[2 * C + 2]
        sid = lax.axis_index("subcore")
        cid = lax.axis_index("core")

        @pl.loop(0, 2)
        def _(j):
            c = j * 16 + sid

            @pl.when(c < NCH)
            def _():
                pltpu.sync_copy(z_hbm, acc.at[pl.ds(c * RC, RC), :])

        plsc.subcore_barrier()

        def body(m_vmem, i_vmem):
            pltpu.sync_copy(m_vmem, acc.at[i_vmem.at[0]], add=True)

        for c in range(C):
            pltpu.emit_pipeline(
                body,
                grid=(EC // GW,),
                in_specs=[
                    pl.BlockSpec((GW, H), lambda i: (i, 0)),
                    pl.BlockSpec((1, GW), lambda i: (0, i)),
                ],
                out_specs=[],
                core_axis_name=("core", "subcore"),
                dimension_semantics=(pltpu.PARALLEL,),
            )(m_refs[c], i_refs[c])
        plsc.subcore_barrier()

        @pl.loop(0, 2)
        def _(j):
            c = j * 16 + sid

            @pl.when(c < NCH)
            def _():
                pltpu.sync_copy(acc.at[pl.ds(c * RC, RC), :],
                                o_hbm.at[cid, pl.ds(c * RC, RC), :])

    return k(*msg_chunks, *idx_chunks, zrows)


def _tc_combine(h, parts):
    def body(h_ref, p_ref, o_ref):
        o_ref[...] = h_ref[...] + DT * (p_ref[0] + p_ref[1])

    return pl.pallas_call(
        body,
        grid=(10,),
        in_specs=[
            pl.BlockSpec((N // 10, H), lambda i: (i, 0)),
            pl.BlockSpec((2, N // 10, H), lambda i: (0, i, 0)),
        ],
        out_specs=pl.BlockSpec((N // 10, H), lambda i: (i, 0)),
        out_shape=jax.ShapeDtypeStruct((N, H), jnp.float32),
    )(h, parts)


def kernel(h, edge_index, v_w1, v_b1, v_w2, v_b2, p_w1, p_b1, p_w2, p_b2,
           f_w1, f_b1, f_w2, f_b2):
    rowc = edge_index[0].reshape(C, EC)
    colc = edge_index[1].reshape(C, EC)
    idx_pair = jnp.stack([rowc, colc], axis=1)

    W1 = jnp.concatenate([v_w1.T, p_w1.T, f_w1.T],
                         axis=1).astype(jnp.bfloat16)
    b1 = jnp.concatenate([v_b1, p_b1, f_b1]).reshape(1, 384)
    W2bd = jnp.zeros((256, 256), jnp.float32)
    W2bd = W2bd.at[:128, :128].set(p_w2.T).at[128:, 128:].set(f_w2.T)
    W2bd = W2bd.astype(jnp.bfloat16)
    b2pf = jnp.concatenate([p_b2, f_b2]).reshape(1, 256)
    w2v = v_w2.reshape(1, 128)
    b2v = v_b2.reshape(1, 1)
    zrows = jnp.zeros((RC, H), jnp.float32)

    msg_chunks = []
    idx_chunks = []
    for c in range(C):
        gath = _sc_gather(h, idx_pair[c].reshape(1, 2 * EC))
        msg_chunks.append(_tc_mlp(gath, W1, b1, w2v, b2v, W2bd, b2pf))
        idx_chunks.append(rowc[c:c + 1, :])

    parts = _sc_scatter(msg_chunks, idx_chunks, zrows)
    return _tc_combine(h, parts)


# speedup vs baseline: 2.5191x; 1.0670x over previous
import functools

import jax
import jax.numpy as jnp
from jax import lax
from jax.experimental import pallas as pl
from jax.experimental.pallas import tpu as pltpu
from jax.experimental.pallas import tpu_sc as plsc

N = 10000
E = 320000
H = 128
DT = 0.03

C = 4
EC = E // C
GW = 128
BE = 3200
RC = 400
NCH = N // RC

_MESH = dict(core_axis_name="core", subcore_axis_name="subcore")


def _sc_gather(h, idx_flat):
    M = idx_flat.shape[1]
    mesh = plsc.VectorSubcoreMesh(**_MESH)

    @functools.partial(
        pl.kernel,
        out_type=jax.ShapeDtypeStruct((M, H), jnp.float32),
        mesh=mesh,
        scratch_types=[pltpu.VMEM_SHARED((N, H), jnp.float32)],
    )
    def k(h_hbm, i_hbm, o_hbm, hs):
        sid = lax.axis_index("subcore")

        @pl.loop(0, 2)
        def _(j):
            c = j * 16 + sid

            @pl.when(c < NCH)
            def _():
                pltpu.sync_copy(h_hbm.at[pl.ds(c * RC, RC), :],
                                hs.at[pl.ds(c * RC, RC), :])

        plsc.subcore_barrier()

        def body(i_vmem, o_vmem):
            pltpu.sync_copy(hs.at[i_vmem.at[0]], o_vmem)

        pltpu.emit_pipeline(
            body,
            grid=(M // GW,),
            in_specs=[pl.BlockSpec((1, GW), lambda i: (0, i))],
            out_specs=[pl.BlockSpec((GW, H), lambda i: (i, 0))],
            core_axis_name=("core", "subcore"),
            dimension_semantics=(pltpu.PARALLEL,),
        )(i_hbm, o_hbm)

    return k(h, idx_flat)


def _tc_mlp(gath, W1, b1, w2v, b2v, W2bd, b2pf):
    nblk = EC // BE
    coff = nblk

    def body(xr_ref, xc_ref, W1_ref, b1_ref, w2v_ref, b2v_ref, W2_ref,
             b2pf_ref, o_ref):
        xr = xr_ref[...]
        xc = xc_ref[...]
        xrb = xr.astype(jnp.bfloat16)
        xcb = xc.astype(jnp.bfloat16)
        z = (jnp.dot(xrb, W1_ref[:128],
                     preferred_element_type=jnp.float32)
             + jnp.dot(xcb, W1_ref[128:],
                       preferred_element_type=jnp.float32) + b1_ref[...])
        t_vp = jnp.tanh(z[:, :256])
        t_f = jnp.maximum(z[:, 256:], 0.0)
        nu = (jnp.sum(t_vp[:, :128] * w2v_ref[...], axis=1, keepdims=True)
              + b2v_ref[0, 0])
        t_p = t_vp[:, 128:].astype(jnp.bfloat16)
        t_fb = t_f.astype(jnp.bfloat16)
        pres = jnp.dot(t_p, W2_ref[:128, :128],
                       preferred_element_type=jnp.float32)
        forc = jnp.dot(t_fb, W2_ref[128:, 128:],
                       preferred_element_type=jnp.float32)
        o_ref[...] = (nu * (xc - xr) + forc - pres
                      + (b2pf_ref[:, 128:] - b2pf_ref[:, :128]))

    full = lambda shape: pl.BlockSpec(shape, lambda i: (0, 0))
    return pl.pallas_call(
        body,
        grid=(nblk,),
        in_specs=[
            pl.BlockSpec((BE, H), lambda i: (i, 0)),
            pl.BlockSpec((BE, H), lambda i: (i + coff, 0)),
            full((256, 384)),
            full((1, 384)),
            full((1, 128)),
            full((1, 1)),
            full((256, 256)),
            full((1, 256)),
        ],
        out_specs=pl.BlockSpec((BE, H), lambda i: (i, 0)),
        out_shape=jax.ShapeDtypeStruct((EC, H), jnp.float32),
    )(gath, gath, W1, b1, w2v, b2v, W2bd, b2pf)


def _sc_scatter(msg_chunks, idx_chunks, zrows):
    mesh = plsc.VectorSubcoreMesh(**_MESH)

    @functools.partial(
        pl.kernel,
        out_type=jax.ShapeDtypeStruct((2, N, H), jnp.float32),
        mesh=mesh,
        scratch_types=[pltpu.VMEM_SHARED((N, H), jnp.float32)],
    )
    def k(*refs):
        m_refs = refs[:C]
        i_refs = refs[C:2 * C]
        z_hbm = refs[2 * C]
        o_hbm = refs[2 * C + 1]
        acc = refs[2 * C + 2]
        sid = lax.axis_index("subcore")
        cid = lax.axis_index("core")

        @pl.loop(0, 2)
        def _(j):
            c = j * 16 + sid

            @pl.when(c < NCH)
            def _():
                pltpu.sync_copy(z_hbm, acc.at[pl.ds(c * RC, RC), :])

        plsc.subcore_barrier()

        def body(m_vmem, i_vmem):
            pltpu.sync_copy(m_vmem, acc.at[i_vmem.at[0]], add=True)

        for c in range(C):
            pltpu.emit_pipeline(
                body,
                grid=(EC // GW,),
                in_specs=[
                    pl.BlockSpec((GW, H), lambda i: (i, 0)),
                    pl.BlockSpec((1, GW), lambda i: (0, i)),
                ],
                out_specs=[],
                core_axis_name=("core", "subcore"),
                dimension_semantics=(pltpu.PARALLEL,),
            )(m_refs[c], i_refs[c])
        plsc.subcore_barrier()

        @pl.loop(0, 2)
        def _(j):
            c = j * 16 + sid

            @pl.when(c < NCH)
            def _():
                pltpu.sync_copy(acc.at[pl.ds(c * RC, RC), :],
                                o_hbm.at[cid, pl.ds(c * RC, RC), :])

    return k(*msg_chunks, *idx_chunks, zrows)


def _tc_combine(h, parts):
    def body(h_ref, p_ref, o_ref):
        o_ref[...] = h_ref[...] + DT * (p_ref[0] + p_ref[1])

    return pl.pallas_call(
        body,
        grid=(10,),
        in_specs=[
            pl.BlockSpec((N // 10, H), lambda i: (i, 0)),
            pl.BlockSpec((2, N // 10, H), lambda i: (0, i, 0)),
        ],
        out_specs=pl.BlockSpec((N // 10, H), lambda i: (i, 0)),
        out_shape=jax.ShapeDtypeStruct((N, H), jnp.float32),
    )(h, parts)


def kernel(h, edge_index, v_w1, v_b1, v_w2, v_b2, p_w1, p_b1, p_w2, p_b2,
           f_w1, f_b1, f_w2, f_b2):
    rowc = edge_index[0].reshape(C, EC)
    colc = edge_index[1].reshape(C, EC)
    idx_pair = jnp.stack([rowc, colc], axis=1)

    W1 = jnp.concatenate([v_w1.T, p_w1.T, f_w1.T],
                         axis=1).astype(jnp.bfloat16)
    b1 = jnp.concatenate([v_b1, p_b1, f_b1]).reshape(1, 384)
    W2bd = jnp.zeros((256, 256), jnp.float32)
    W2bd = W2bd.at[:128, :128].set(p_w2.T).at[128:, 128:].set(f_w2.T)
    W2bd = W2bd.astype(jnp.bfloat16)
    b2pf = jnp.concatenate([p_b2, f_b2]).reshape(1, 256)
    w2v = v_w2.reshape(1, 128)
    b2v = v_b2.reshape(1, 1)
    zrows = jnp.zeros((RC, H), jnp.float32)

    msg_chunks = []
    idx_chunks = []
    for c in range(C):
        gath = _sc_gather(h, idx_pair[c].reshape(1, 2 * EC))
        msg_chunks.append(_tc_mlp(gath, W1, b1, w2v, b2v, W2bd, b2pf))
        idx_chunks.append(rowc[c:c + 1, :])

    parts = _sc_scatter(msg_chunks, idx_chunks, zrows)
    return _tc_combine(h, parts)


# speedup vs baseline: 2.7696x; 1.0995x over previous
import functools

import jax
import jax.numpy as jnp
from jax import lax
from jax.experimental import pallas as pl
from jax.experimental.pallas import tpu as pltpu
from jax.experimental.pallas import tpu_sc as plsc

N = 10000
E = 320000
H = 128
DT = 0.03

C = 4
EC = E // C
GW = 128
BE = 3200
RC = 400
NCH = N // RC

_MESH = dict(core_axis_name="core", subcore_axis_name="subcore")


def _sc_gather(h, ridx, cidx):
    mesh = plsc.VectorSubcoreMesh(**_MESH)

    @functools.partial(
        pl.kernel,
        out_type=(jax.ShapeDtypeStruct((EC, H), jnp.float32),
                  jax.ShapeDtypeStruct((EC, H), jnp.float32)),
        mesh=mesh,
        scratch_types=[pltpu.VMEM_SHARED((N, H), jnp.float32)],
    )
    def k(h_hbm, ir_hbm, ic_hbm, or_hbm, oc_hbm, hs):
        sid = lax.axis_index("subcore")

        @pl.loop(0, 2)
        def _(j):
            c = j * 16 + sid

            @pl.when(c < NCH)
            def _():
                pltpu.sync_copy(h_hbm.at[pl.ds(c * RC, RC), :],
                                hs.at[pl.ds(c * RC, RC), :])

        plsc.subcore_barrier()

        def body(i_v, o_v):
            pltpu.sync_copy(hs.at[i_v.at[0]], o_v)

        for i_hbm, o_hbm in ((ir_hbm, or_hbm), (ic_hbm, oc_hbm)):
            pltpu.emit_pipeline(
                body,
                grid=(EC // GW,),
                in_specs=[pl.BlockSpec((1, GW), lambda i: (0, i))],
                out_specs=[pl.BlockSpec((GW, H), lambda i: (i, 0))],
                core_axis_name=("core", "subcore"),
                dimension_semantics=(pltpu.PARALLEL,),
            )(i_hbm, o_hbm)

    return k(h, ridx, cidx)


def _tc_mlp(hrow, hcol, W1, b1, w2v, b2v, W2bd, b2pf):
    nblk = EC // BE

    def body(xr_ref, xc_ref, W1_ref, b1_ref, w2v_ref, b2v_ref, W2_ref,
             b2pf_ref, o_ref):
        xr = xr_ref[...]
        xc = xc_ref[...]
        xrb = xr.astype(jnp.bfloat16)
        xcb = xc.astype(jnp.bfloat16)
        z = (jnp.dot(xrb, W1_ref[:128],
                     preferred_element_type=jnp.float32)
             + jnp.dot(xcb, W1_ref[128:],
                       preferred_element_type=jnp.float32) + b1_ref[...])
        t_vp = jnp.tanh(z[:, :256])
        t_f = jnp.maximum(z[:, 256:], 0.0)
        nu = (jnp.sum(t_vp[:, :128] * w2v_ref[...], axis=1, keepdims=True)
              + b2v_ref[0, 0])
        t_p = t_vp[:, 128:].astype(jnp.bfloat16)
        t_fb = t_f.astype(jnp.bfloat16)
        pres = jnp.dot(t_p, W2_ref[:128, :128],
                       preferred_element_type=jnp.float32)
        forc = jnp.dot(t_fb, W2_ref[128:, 128:],
                       preferred_element_type=jnp.float32)
        o_ref[...] = (nu * (xc - xr) + forc - pres
                      + (b2pf_ref[:, 128:] - b2pf_ref[:, :128]))

    full = lambda shape: pl.BlockSpec(shape, lambda i: (0, 0))
    return pl.pallas_call(
        body,
        grid=(nblk,),
        in_specs=[
            pl.BlockSpec((BE, H), lambda i: (i, 0)),
            pl.BlockSpec((BE, H), lambda i: (i, 0)),
            full((256, 384)),
            full((1, 384)),
            full((1, 128)),
            full((1, 1)),
            full((256, 256)),
            full((1, 256)),
        ],
        out_specs=pl.BlockSpec((BE, H), lambda i: (i, 0)),
        out_shape=jax.ShapeDtypeStruct((EC, H), jnp.float32),
    )(hrow, hcol, W1, b1, w2v, b2v, W2bd, b2pf)


def _sc_scatter(msg_chunks, idx_chunks, zrows):
    mesh = plsc.VectorSubcoreMesh(**_MESH)
    nm = len(msg_chunks)

    @functools.partial(
        pl.kernel,
        out_type=jax.ShapeDtypeStruct((2, N, H), jnp.float32),
        mesh=mesh,
        scratch_types=[pltpu.VMEM_SHARED((N, H), jnp.float32)],
    )
    def k(*refs):
        m_refs = refs[:nm]
        i_refs = refs[nm:2 * nm]
        z_hbm = refs[2 * nm]
        o_hbm = refs[2 * nm + 1]
        acc = refs[2 * nm + 2]
        sid = lax.axis_index("subcore")
        cid = lax.axis_index("core")

        @pl.loop(0, 2)
        def _(j):
            c = j * 16 + sid

            @pl.when(c < NCH)
            def _():
                pltpu.sync_copy(z_hbm, acc.at[pl.ds(c * RC, RC), :])

        plsc.subcore_barrier()

        def body(m_vmem, i_vmem):
            pltpu.sync_copy(m_vmem, acc.at[i_vmem.at[0]], add=True)

        for c in range(nm):
            pltpu.emit_pipeline(
                body,
                grid=(EC // GW,),
                in_specs=[
                    pl.BlockSpec((GW, H), lambda i: (i, 0)),
                    pl.BlockSpec((1, GW), lambda i: (0, i)),
                ],
                out_specs=[],
                core_axis_name=("core", "subcore"),
                dimension_semantics=(pltpu.PARALLEL,),
            )(m_refs[c], i_refs[c])
        plsc.subcore_barrier()

        @pl.loop(0, 2)
        def _(j):
            c = j * 16 + sid

            @pl.when(c < NCH)
            def _():
                pltpu.sync_copy(acc.at[pl.ds(c * RC, RC), :],
                                o_hbm.at[cid, pl.ds(c * RC, RC), :])

    return k(*msg_chunks, *idx_chunks, zrows)


def _tc_combine(h, parts_a, parts_b):
    def body(h_ref, pa_ref, pb_ref, o_ref):
        o_ref[...] = h_ref[...] + DT * ((pa_ref[0] + pa_ref[1])
                                        + (pb_ref[0] + pb_ref[1]))

    pspec = pl.BlockSpec((2, N // 10, H), lambda i: (0, i, 0))
    return pl.pallas_call(
        body,
        grid=(10,),
        in_specs=[
            pl.BlockSpec((N // 10, H), lambda i: (i, 0)),
            pspec,
            pspec,
        ],
        out_specs=pl.BlockSpec((N // 10, H), lambda i: (i, 0)),
        out_shape=jax.ShapeDtypeStruct((N, H), jnp.float32),
    )(h, parts_a, parts_b)


def kernel(h, edge_index, v_w1, v_b1, v_w2, v_b2, p_w1, p_b1, p_w2, p_b2,
           f_w1, f_b1, f_w2, f_b2):
    W1 = jnp.concatenate([v_w1.T, p_w1.T, f_w1.T],
                         axis=1).astype(jnp.bfloat16)
    b1 = jnp.concatenate([v_b1, p_b1, f_b1]).reshape(1, 384)
    W2bd = jnp.zeros((256, 256), jnp.float32)
    W2bd = W2bd.at[:128, :128].set(p_w2.T).at[128:, 128:].set(f_w2.T)
    W2bd = W2bd.astype(jnp.bfloat16)
    b2pf = jnp.concatenate([p_b2, f_b2]).reshape(1, 256)
    w2v = v_w2.reshape(1, 128)
    b2v = v_b2.reshape(1, 1)
    zrows = jnp.zeros((RC, H), jnp.float32)

    msg_chunks = []
    idx_chunks = []
    for c in range(C):
        ridx = edge_index[0:1, c * EC:(c + 1) * EC]
        cidx = edge_index[1:2, c * EC:(c + 1) * EC]
        hrow, hcol = _sc_gather(h, ridx, cidx)
        msg_chunks.append(_tc_mlp(hrow, hcol, W1, b1, w2v, b2v, W2bd, b2pf))
        idx_chunks.append(ridx)

    parts_a = _sc_scatter(msg_chunks[:2], idx_chunks[:2], zrows)
    parts_b = _sc_scatter(msg_chunks[2:], idx_chunks[2:], zrows)
    return _tc_combine(h, parts_a, parts_b)
